# Initial kernel scaffold; baseline (speedup 1.0000x reference)
import jax
import jax.numpy as jnp
from jax.experimental import pallas as pl


def kernel(x, c, edge_index_xx, edge_index_cx, W_self, b_self, W_cc, b_cc, W_edge, b_edge, W_cx, b_cx, W_pool, b_pool):
    raise NotImplementedError("write your pallas kernel here")


# speedup vs baseline: 59.0353x; 59.0353x over previous
import functools

import jax
import jax.numpy as jnp
from jax import lax
from jax.experimental import pallas as pl
from jax.experimental.pallas import tpu as pltpu
from jax.experimental.pallas import tpu_sc as plsc

_N = 10000
_NC = 2048
_EXX = 320000
_ECX = 100000
_D = 128

_NTILE = 16
_NW = 32
_N_PAD = 10240
_NC_PAD = 2304
_ECX_PAD = 100352
_EPT_XX = _EXX // _NTILE
_EPT_CX = _ECX_PAD // _NTILE
_NPT_XX = _N_PAD // _NTILE
_NPT_CX = _NC_PAD // _NTILE
_ROWS_XX = _N_PAD // _NW
_ROWS_CX = _NC_PAD // _NW

_sc_mesh = plsc.VectorSubcoreMesh(core_axis_name="c", subcore_axis_name="s")
_sc_params = pltpu.CompilerParams(needs_layout_passes=False)


@functools.partial(
    pl.kernel,
    out_type=[
        jax.ShapeDtypeStruct((_N_PAD,), jnp.float32),
        jax.ShapeDtypeStruct((_N_PAD,), jnp.int32),
        jax.ShapeDtypeStruct((_NC_PAD,), jnp.float32),
        jax.ShapeDtypeStruct((_NC_PAD,), jnp.int32),
    ],
    mesh=_sc_mesh,
    scratch_types=[
        pltpu.VMEM((_EPT_XX,), jnp.int32),
        pltpu.VMEM((_EPT_XX,), jnp.int32),
        pltpu.VMEM((_N_PAD,), jnp.float32),
        pltpu.VMEM((_N_PAD,), jnp.int32),
        pltpu.VMEM((_NTILE * _NPT_XX,), jnp.float32),
        pltpu.VMEM((_NTILE * _NPT_XX,), jnp.int32),
        pltpu.VMEM_SHARED((_NTILE * _N_PAD,), jnp.float32),
        pltpu.VMEM_SHARED((_NTILE * _N_PAD,), jnp.int32),
    ],
    compiler_params=_sc_params,
)
def _edge_stats(src_xx, dst_xx, src_cx, dst_cx,
                cnt_xx_out, wsrc_xx_out, cnt_cx_out, wsrc_cx_out,
                src_v, dst_v, cnt_v, win_v, ccnt_v, cwin_v, s_cnt, s_win):
    cid = lax.axis_index("c")
    sid = lax.axis_index("s")

    def run(src_hbm, dst_hbm, cnt_out, wsrc_out, ept, n_pad, npt):
        nvec_n = n_pad // 16
        minus1 = jnp.full((16,), -1, jnp.int32)
        zf = jnp.zeros((16,), jnp.float32)

        def initb(i, carry):
            cnt_v[pl.ds(i * 16, 16)] = zf
            win_v[pl.ds(i * 16, 16)] = minus1
            return carry

        lax.fori_loop(0, nvec_n, initb, 0, unroll=4)

        base = sid * ept
        pltpu.sync_copy(src_hbm.at[pl.ds(base, ept)], src_v.at[pl.ds(0, ept)])
        pltpu.sync_copy(dst_hbm.at[pl.ds(base, ept)], dst_v.at[pl.ds(0, ept)])

        ones = jnp.ones((16,), jnp.float32)

        def edgeb(i, carry):
            d16 = dst_v[pl.ds(i * 16, 16)]
            s16 = src_v[pl.ds(i * 16, 16)]
            plsc.store_scatter(win_v, [d16], s16)
            plsc.addupdate_scatter(cnt_v, [d16], ones)
            return carry

        lax.fori_loop(0, ept // 16, edgeb, 0)

        tkey = sid << 14

        def convb(i, carry):
            w = win_v[pl.ds(i * 16, 16)]
            win_v[pl.ds(i * 16, 16)] = jnp.where(w < 0, w, w | tkey)
            return carry

        lax.fori_loop(0, nvec_n, convb, 0, unroll=4)

        pltpu.sync_copy(cnt_v.at[pl.ds(0, n_pad)], s_cnt.at[pl.ds(sid * n_pad, n_pad)])
        pltpu.sync_copy(win_v.at[pl.ds(0, n_pad)], s_win.at[pl.ds(sid * n_pad, n_pad)])
        plsc.subcore_barrier()

        nbase = sid * npt
        for k in range(_NTILE):
            pltpu.sync_copy(s_cnt.at[pl.ds(k * n_pad + nbase, npt)],
                            ccnt_v.at[pl.ds(k * npt, npt)])
            pltpu.sync_copy(s_win.at[pl.ds(k * n_pad + nbase, npt)],
                            cwin_v.at[pl.ds(k * npt, npt)])

        def combb(j, carry):
            acc_c = ccnt_v[pl.ds(j * 16, 16)]
            acc_w = cwin_v[pl.ds(j * 16, 16)]
            for k in range(1, _NTILE):
                acc_c = acc_c + ccnt_v[pl.ds(k * npt + j * 16, 16)]
                acc_w = jnp.maximum(acc_w, cwin_v[pl.ds(k * npt + j * 16, 16)])
            cnt_v[pl.ds(j * 16, 16)] = acc_c
            win_v[pl.ds(j * 16, 16)] = jnp.where(acc_w < 0, 0, acc_w & 16383)
            return carry

        lax.fori_loop(0, npt // 16, combb, 0)

        pltpu.sync_copy(cnt_v.at[pl.ds(0, npt)], cnt_out.at[pl.ds(nbase, npt)])
        pltpu.sync_copy(win_v.at[pl.ds(0, npt)], wsrc_out.at[pl.ds(nbase, npt)])

    @pl.when(cid == 0)
    def _():
        run(src_xx, dst_xx, cnt_xx_out, wsrc_xx_out, _EPT_XX, _N_PAD, _NPT_XX)

    @pl.when(cid == 1)
    def _():
        run(src_cx, dst_cx, cnt_cx_out, wsrc_cx_out, _EPT_CX, _NC_PAD, _NPT_CX)


@functools.partial(
    pl.kernel,
    out_type=[
        jax.ShapeDtypeStruct((_N_PAD, _D), jnp.float32),
        jax.ShapeDtypeStruct((_NC_PAD, _D), jnp.float32),
    ],
    mesh=_sc_mesh,
    scratch_types=[
        pltpu.VMEM((_ROWS_XX,), jnp.int32),
        pltpu.VMEM((_ROWS_XX,), jnp.float32),
        pltpu.VMEM((_ROWS_XX, _D), jnp.float32),
        pltpu.VMEM((_ROWS_CX,), jnp.int32),
        pltpu.VMEM((_ROWS_CX,), jnp.float32),
        pltpu.VMEM((_ROWS_CX, _D), jnp.float32),
        pltpu.SemaphoreType.DMA,
    ],
    compiler_params=_sc_params,
)
def _gather_rows(phi_hbm, psi_hbm, wsrc_xx, cnt_xx, wsrc_cx, cnt_cx,
                 gphi_out, gpsi_out,
                 idx_v, cntx_v, rows_v, idx2_v, cntc_v, rows2_v, sem):
    cid = lax.axis_index("c")
    sid = lax.axis_index("s")
    wid = sid * 2 + cid

    def scale_rows(rows, cnts, nrows):
        def rowb(r, carry):
            s16 = plsc.load_gather(cnts, [lax.broadcast(r, (16,))])
            for j in range(_D // 16):
                rows[r, pl.ds(j * 16, 16)] = rows[r, pl.ds(j * 16, 16)] * s16
            return carry
        lax.fori_loop(0, nrows, rowb, 0)

    base = wid * _ROWS_XX
    pltpu.sync_copy(wsrc_xx.at[pl.ds(base, _ROWS_XX)], idx_v)
    pltpu.sync_copy(cnt_xx.at[pl.ds(base, _ROWS_XX)], cntx_v)
    for off, ln in ((0, 128), (128, 128), (256, 64)):
        pltpu.async_copy(phi_hbm.at[idx_v.at[pl.ds(off, ln)]],
                         rows_v.at[pl.ds(off, ln)], sem).wait()
    scale_rows(rows_v, cntx_v, _ROWS_XX)
    pltpu.sync_copy(rows_v, gphi_out.at[pl.ds(base, _ROWS_XX)])

    base2 = wid * _ROWS_CX
    pltpu.sync_copy(wsrc_cx.at[pl.ds(base2, _ROWS_CX)], idx2_v)
    pltpu.sync_copy(cnt_cx.at[pl.ds(base2, _ROWS_CX)], cntc_v)
    pltpu.async_copy(psi_hbm.at[idx2_v], rows2_v, sem).wait()
    scale_rows(rows2_v, cntc_v, _ROWS_CX)
    pltpu.sync_copy(rows2_v, gpsi_out.at[pl.ds(base2, _ROWS_CX)])


def _phi_body(x_ref, w_ref, b_ref, o_ref):
    acc = jnp.dot(x_ref[...], w_ref[...], preferred_element_type=jnp.float32)
    o_ref[...] = jnp.maximum(acc + b_ref[...], 0.0)


def _c_body(c_ref, wcx_ref, bcx_ref, wcc_ref, bcc_ref, psi_ref, outc_ref):
    cc = c_ref[...]
    psi_ref[...] = jnp.maximum(
        jnp.dot(cc, wcx_ref[...], preferred_element_type=jnp.float32) + bcx_ref[...], 0.0)
    outc_ref[...] = jnp.maximum(
        jnp.dot(cc, wcc_ref[...], preferred_element_type=jnp.float32) + bcc_ref[...], 0.0)


def _out_body(x_ref, ws_ref, bs_ref, gphi_ref, gpsi_ref, wp_ref, bp_ref, o_ref):
    i = pl.program_id(0)
    a = jnp.maximum(
        jnp.dot(x_ref[...], ws_ref[...], preferred_element_type=jnp.float32) + bs_ref[...],
        0.0)
    a = a + gphi_ref[...]
    a = a + jnp.where(i == 0, 1.0, 0.0) * gpsi_ref[...]
    o_ref[...] = jnp.dot(a, wp_ref[...], preferred_element_type=jnp.float32) + bp_ref[...]


_BLK = 2048
_GRID = 5


def kernel(x, c, edge_index_xx, edge_index_cx, W_self, b_self, W_cc, b_cc,
           W_edge, b_edge, W_cx, b_cx, W_pool, b_pool):
    src_xx = edge_index_xx[0]
    dst_xx = edge_index_xx[1]
    pad = _ECX_PAD - _ECX
    src_cx = jnp.concatenate([edge_index_cx[0], jnp.zeros((pad,), jnp.int32)])
    dst_cx = jnp.concatenate(
        [edge_index_cx[1], jnp.full((pad,), _NC_PAD - 1, jnp.int32)])

    cnt_xx, wsrc_xx, cnt_cx, wsrc_cx = _edge_stats(src_xx, dst_xx, src_cx, dst_cx)

    phi = pl.pallas_call(
        _phi_body,
        grid=(_GRID,),
        in_specs=[pl.BlockSpec((_BLK, _D), lambda i: (i, 0)),
                  pl.BlockSpec((_D, _D), lambda i: (0, 0)),
                  pl.BlockSpec((1, _D), lambda i: (0, 0))],
        out_specs=pl.BlockSpec((_BLK, _D), lambda i: (i, 0)),
        out_shape=jax.ShapeDtypeStruct((_N, _D), jnp.float32),
    )(x, W_edge, b_edge.reshape(1, _D))

    psi, out_c = pl.pallas_call(
        _c_body,
        out_shape=[jax.ShapeDtypeStruct((_NC, _D), jnp.float32),
                   jax.ShapeDtypeStruct((_NC, _D), jnp.float32)],
    )(c, W_cx, b_cx.reshape(1, _D), W_cc, b_cc.reshape(1, _D))

    gphi, gpsi = _gather_rows(phi, psi, wsrc_xx, cnt_xx, wsrc_cx, cnt_cx)

    out_x = pl.pallas_call(
        _out_body,
        grid=(_GRID,),
        in_specs=[pl.BlockSpec((_BLK, _D), lambda i: (i, 0)),
                  pl.BlockSpec((_D, _D), lambda i: (0, 0)),
                  pl.BlockSpec((1, _D), lambda i: (0, 0)),
                  pl.BlockSpec((_BLK, _D), lambda i: (i, 0)),
                  pl.BlockSpec((_BLK, _D), lambda i: (0, 0)),
                  pl.BlockSpec((_D, _D), lambda i: (0, 0)),
                  pl.BlockSpec((1, _D), lambda i: (0, 0))],
        out_specs=pl.BlockSpec((_BLK, _D), lambda i: (i, 0)),
        out_shape=jax.ShapeDtypeStruct((_N, _D), jnp.float32),
    )(x, W_self, b_self.reshape(1, _D), gphi, gpsi, W_pool, b_pool.reshape(1, _D))

    return (out_x, out_c)


# speedup vs baseline: 69.1150x; 1.1707x over previous
import functools

import jax
import jax.numpy as jnp
from jax import lax
from jax.experimental import pallas as pl
from jax.experimental.pallas import tpu as pltpu
from jax.experimental.pallas import tpu_sc as plsc

_N = 10000
_NC = 2048
_EXX = 320000
_ECX = 100000
_D = 128

_NTILE = 16
_NW = 32
_N_PAD = 10240
_NC_PAD = 2304
_ECX_PAD = 100352
_EPT_XX = _EXX // _NTILE
_EPT_CX = _ECX_PAD // _NTILE
_NPT_XX = _N_PAD // _NTILE
_NPT_CX = _NC_PAD // _NTILE
_ROWS_XX = _N_PAD // _NW
_ROWS_CX = _NC_PAD // _NW

_sc_mesh = plsc.VectorSubcoreMesh(core_axis_name="c", subcore_axis_name="s")
_sc_params = pltpu.CompilerParams(needs_layout_passes=False)


@functools.partial(
    pl.kernel,
    out_type=[
        jax.ShapeDtypeStruct((_N_PAD,), jnp.float32),
        jax.ShapeDtypeStruct((_N_PAD,), jnp.int32),
        jax.ShapeDtypeStruct((_NC_PAD,), jnp.float32),
        jax.ShapeDtypeStruct((_NC_PAD,), jnp.int32),
    ],
    mesh=_sc_mesh,
    scratch_types=[
        pltpu.VMEM((_EPT_XX,), jnp.int32),
        pltpu.VMEM((_EPT_XX,), jnp.int32),
        pltpu.VMEM((_N_PAD,), jnp.float32),
        pltpu.VMEM((_N_PAD,), jnp.int32),
        pltpu.VMEM((_NTILE * _NPT_XX,), jnp.float32),
        pltpu.VMEM((_NTILE * _NPT_XX,), jnp.int32),
        pltpu.VMEM_SHARED((_NTILE * _N_PAD,), jnp.float32),
        pltpu.VMEM_SHARED((_NTILE * _N_PAD,), jnp.int32),
        pltpu.SemaphoreType.DMA,
    ],
    compiler_params=_sc_params,
)
def _edge_stats(src_xx, dst_xx, src_cx, dst_cx,
                cnt_xx_out, wsrc_xx_out, cnt_cx_out, wsrc_cx_out,
                src_v, dst_v, cnt_v, win_v, ccnt_v, cwin_v, s_cnt, s_win, sem):
    cid = lax.axis_index("c")
    sid = lax.axis_index("s")

    def run(src_hbm, dst_hbm, cnt_out, wsrc_out, ept, n_pad, npt):
        nvec_n = n_pad // 16
        minus1 = jnp.full((16,), -1, jnp.int32)
        zf = jnp.zeros((16,), jnp.float32)

        base = sid * ept
        eloads = [pltpu.make_async_copy(src_hbm.at[pl.ds(base, ept)],
                                        src_v.at[pl.ds(0, ept)], sem),
                  pltpu.make_async_copy(dst_hbm.at[pl.ds(base, ept)],
                                        dst_v.at[pl.ds(0, ept)], sem)]
        for cp in eloads:
            cp.start()

        def initb(i, carry):
            cnt_v[pl.ds(i * 16, 16)] = zf
            win_v[pl.ds(i * 16, 16)] = minus1
            return carry

        lax.fori_loop(0, nvec_n, initb, 0, unroll=4)

        for cp in eloads:
            cp.wait()

        ones = jnp.ones((16,), jnp.float32)

        grp = 10 if ept == _EPT_XX else 8
        assert (ept // 16) % grp == 0

        def edgeb(i, carry):
            b = i * (16 * grp)
            ds = [dst_v[pl.ds(b + k * 16, 16)] for k in range(grp)]
            ss = [src_v[pl.ds(b + k * 16, 16)] for k in range(grp)]
            for k in range(grp):
                plsc.store_scatter(win_v, [ds[k]], ss[k])
                plsc.addupdate_scatter(cnt_v, [ds[k]], ones)
            return carry

        lax.fori_loop(0, ept // (16 * grp), edgeb, 0)

        tkey = sid << 14

        cstage = pltpu.make_async_copy(cnt_v.at[pl.ds(0, n_pad)],
                                       s_cnt.at[pl.ds(sid * n_pad, n_pad)], sem)
        cstage.start()

        def convb(i, carry):
            b = i * 64
            ws = [win_v[pl.ds(b + k * 16, 16)] for k in range(4)]
            for k in range(4):
                win_v[pl.ds(b + k * 16, 16)] = jnp.where(ws[k] < 0, ws[k], ws[k] | tkey)
            return carry

        lax.fori_loop(0, nvec_n // 4, convb, 0)

        wstage = pltpu.make_async_copy(win_v.at[pl.ds(0, n_pad)],
                                       s_win.at[pl.ds(sid * n_pad, n_pad)], sem)
        wstage.start()
        cstage.wait()
        wstage.wait()
        plsc.subcore_barrier()

        nbase = sid * npt
        stages = []
        for k in range(_NTILE):
            stages.append(pltpu.make_async_copy(
                s_cnt.at[pl.ds(k * n_pad + nbase, npt)],
                ccnt_v.at[pl.ds(k * npt, npt)], sem))
            stages.append(pltpu.make_async_copy(
                s_win.at[pl.ds(k * n_pad + nbase, npt)],
                cwin_v.at[pl.ds(k * npt, npt)], sem))
        for cp in stages:
            cp.start()
        for cp in stages:
            cp.wait()

        def combb(j, carry):
            acc_c = ccnt_v[pl.ds(j * 16, 16)]
            acc_w = cwin_v[pl.ds(j * 16, 16)]
            for k in range(1, _NTILE):
                acc_c = acc_c + ccnt_v[pl.ds(k * npt + j * 16, 16)]
                acc_w = jnp.maximum(acc_w, cwin_v[pl.ds(k * npt + j * 16, 16)])
            cnt_v[pl.ds(j * 16, 16)] = acc_c
            win_v[pl.ds(j * 16, 16)] = jnp.where(acc_w < 0, 0, acc_w & 16383)
            return carry

        lax.fori_loop(0, npt // 16, combb, 0)

        pltpu.sync_copy(cnt_v.at[pl.ds(0, npt)], cnt_out.at[pl.ds(nbase, npt)])
        pltpu.sync_copy(win_v.at[pl.ds(0, npt)], wsrc_out.at[pl.ds(nbase, npt)])

    @pl.when(cid == 0)
    def _():
        run(src_xx, dst_xx, cnt_xx_out, wsrc_xx_out, _EPT_XX, _N_PAD, _NPT_XX)

    @pl.when(cid == 1)
    def _():
        run(src_cx, dst_cx, cnt_cx_out, wsrc_cx_out, _EPT_CX, _NC_PAD, _NPT_CX)


@functools.partial(
    pl.kernel,
    out_type=[
        jax.ShapeDtypeStruct((_N_PAD, _D), jnp.float32),
        jax.ShapeDtypeStruct((_NC_PAD, _D), jnp.float32),
    ],
    mesh=_sc_mesh,
    scratch_types=[
        pltpu.VMEM((_ROWS_XX,), jnp.int32),
        pltpu.VMEM((_ROWS_XX, _D), jnp.float32),
        pltpu.VMEM((_ROWS_CX,), jnp.int32),
        pltpu.VMEM((_ROWS_CX, _D), jnp.float32),
        pltpu.SemaphoreType.DMA,
    ],
    compiler_params=_sc_params,
)
def _gather_rows(phi_hbm, psi_hbm, wsrc_xx, wsrc_cx,
                 gphi_out, gpsi_out,
                 idx_v, rows_v, idx2_v, rows2_v, sem):
    cid = lax.axis_index("c")
    sid = lax.axis_index("s")
    wid = sid * 2 + cid

    base = wid * _ROWS_XX
    pltpu.sync_copy(wsrc_xx.at[pl.ds(base, _ROWS_XX)], idx_v)
    base2 = wid * _ROWS_CX
    pltpu.sync_copy(wsrc_cx.at[pl.ds(base2, _ROWS_CX)], idx2_v)
    copies = [pltpu.make_async_copy(phi_hbm.at[idx_v.at[pl.ds(off, 80)]],
                                    rows_v.at[pl.ds(off, 80)], sem)
              for off in (0, 80, 160, 240)]
    copies.append(pltpu.make_async_copy(psi_hbm.at[idx2_v], rows2_v, sem))
    for cp in copies:
        cp.start()
    for cp in copies:
        cp.wait()
    pltpu.sync_copy(rows_v, gphi_out.at[pl.ds(base, _ROWS_XX)])
    pltpu.sync_copy(rows2_v, gpsi_out.at[pl.ds(base2, _ROWS_CX)])


def _phi_body(x_ref, w_ref, b_ref, o_ref):
    acc = jnp.dot(x_ref[...], w_ref[...], preferred_element_type=jnp.float32)
    o_ref[...] = jnp.maximum(acc + b_ref[...], 0.0)


def _c_body(c_ref, wcx_ref, bcx_ref, wcc_ref, bcc_ref, psi_ref, outc_ref):
    cc = c_ref[...]
    psi_ref[...] = jnp.maximum(
        jnp.dot(cc, wcx_ref[...], preferred_element_type=jnp.float32) + bcx_ref[...], 0.0)
    outc_ref[...] = jnp.maximum(
        jnp.dot(cc, wcc_ref[...], preferred_element_type=jnp.float32) + bcc_ref[...], 0.0)


def _out_body(x_ref, ws_ref, bs_ref, cx_ref, gphi_ref, cc_ref, gpsi_ref,
              wp_ref, bp_ref, o_ref):
    i = pl.program_id(0)
    a = jnp.maximum(
        jnp.dot(x_ref[...], ws_ref[...], preferred_element_type=jnp.float32) + bs_ref[...],
        0.0)
    a = a + cx_ref[...] * gphi_ref[...]
    a = a + jnp.where(i == 0, 1.0, 0.0) * (cc_ref[...] * gpsi_ref[...])
    o_ref[...] = jnp.dot(a, wp_ref[...], preferred_element_type=jnp.float32) + bp_ref[...]


_BLK = 2048
_GRID = 5


def kernel(x, c, edge_index_xx, edge_index_cx, W_self, b_self, W_cc, b_cc,
           W_edge, b_edge, W_cx, b_cx, W_pool, b_pool):
    src_xx = edge_index_xx[0]
    dst_xx = edge_index_xx[1]
    pad = _ECX_PAD - _ECX
    src_cx = jnp.concatenate([edge_index_cx[0], jnp.zeros((pad,), jnp.int32)])
    dst_cx = jnp.concatenate(
        [edge_index_cx[1], jnp.full((pad,), _NC_PAD - 1, jnp.int32)])

    cnt_xx, wsrc_xx, cnt_cx, wsrc_cx = _edge_stats(src_xx, dst_xx, src_cx, dst_cx)

    phi = pl.pallas_call(
        _phi_body,
        grid=(_GRID,),
        in_specs=[pl.BlockSpec((_BLK, _D), lambda i: (i, 0)),
                  pl.BlockSpec((_D, _D), lambda i: (0, 0)),
                  pl.BlockSpec((1, _D), lambda i: (0, 0))],
        out_specs=pl.BlockSpec((_BLK, _D), lambda i: (i, 0)),
        out_shape=jax.ShapeDtypeStruct((_N, _D), jnp.float32),
    )(x, W_edge, b_edge.reshape(1, _D))

    psi, out_c = pl.pallas_call(
        _c_body,
        out_shape=[jax.ShapeDtypeStruct((_NC, _D), jnp.float32),
                   jax.ShapeDtypeStruct((_NC, _D), jnp.float32)],
    )(c, W_cx, b_cx.reshape(1, _D), W_cc, b_cc.reshape(1, _D))

    gphi, gpsi = _gather_rows(phi, psi, wsrc_xx, wsrc_cx)

    out_x = pl.pallas_call(
        _out_body,
        grid=(_GRID,),
        in_specs=[pl.BlockSpec((_BLK, _D), lambda i: (i, 0)),
                  pl.BlockSpec((_D, _D), lambda i: (0, 0)),
                  pl.BlockSpec((1, _D), lambda i: (0, 0)),
                  pl.BlockSpec((_BLK, 1), lambda i: (i, 0)),
                  pl.BlockSpec((_BLK, _D), lambda i: (i, 0)),
                  pl.BlockSpec((_BLK, 1), lambda i: (0, 0)),
                  pl.BlockSpec((_BLK, _D), lambda i: (0, 0)),
                  pl.BlockSpec((_D, _D), lambda i: (0, 0)),
                  pl.BlockSpec((1, _D), lambda i: (0, 0))],
        out_specs=pl.BlockSpec((_BLK, _D), lambda i: (i, 0)),
        out_shape=jax.ShapeDtypeStruct((_N, _D), jnp.float32),
    )(x, W_self, b_self.reshape(1, _D), cnt_xx.reshape(_N_PAD, 1), gphi,
      cnt_cx.reshape(_NC_PAD, 1), gpsi, W_pool, b_pool.reshape(1, _D))

    return (out_x, out_c)
